# baseline (speedup 1.0000x reference)
import functools

import jax
import jax.numpy as jnp
from jax import lax
from jax.experimental import pallas as pl
from jax.experimental.pallas import tpu as pltpu
from jax.experimental.pallas import tpu_sc as plsc

N = 10000
E = 320000
D = 128

NC = 2
NS = 16
NW = NC * NS

CH = 128
CPT = 80
EPT = CPT * CH
E_PAD = NW * EPT
N_ACC = 10240
SPAN = N_ACC // NS
DW = 16

_MESH = plsc.VectorSubcoreMesh(
    core_axis_name="c", subcore_axis_name="s", num_cores=NC, num_subcores=NS
)


def _deg_body(dst2d_hbm, degout_hbm, idxbuf, ones_v, zseg, acc):
    c = lax.axis_index("c")
    s = lax.axis_index("s")
    wid = c * NS + s

    def z(k, carry):
        zseg[pl.ds(k * 16, 16)] = jnp.zeros((16,), jnp.float32)
        return carry

    lax.fori_loop(0, SPAN // 16, z, 0)
    pltpu.sync_copy(zseg, acc.at[pl.ds(s * SPAN, SPAN)])

    def o(k, carry):
        ones_v[pl.ds(k * 16, 16)] = jnp.ones((16,), jnp.float32)
        return carry

    lax.fori_loop(0, CH // 16, o, 0)
    pltpu.sync_copy(dst2d_hbm.at[pl.ds(wid * CPT, CPT)], idxbuf)
    plsc.subcore_barrier()

    def body(j, carry):
        pltpu.sync_copy(ones_v, acc.at[idxbuf.at[j]], add=True)
        return carry

    lax.fori_loop(0, CPT, body, 0)
    plsc.subcore_barrier()
    pltpu.sync_copy(
        acc.at[pl.ds(s * SPAN, SPAN)], degout_hbm.at[c, pl.ds(s * SPAN, SPAN)]
    )


def _make_deg_kernel(interpret=False):
    return functools.partial(
        pl.kernel,
        out_type=jax.ShapeDtypeStruct((NC, N_ACC), jnp.float32),
        mesh=_MESH,
        scratch_types=[
            pltpu.VMEM((CPT, CH), jnp.int32),
            pltpu.VMEM((CH,), jnp.float32),
            pltpu.VMEM((SPAN,), jnp.float32),
            pltpu.VMEM_SHARED((N_ACC,), jnp.float32),
        ],
        interpret=interpret,
    )(_deg_body)


_deg_kernel = _make_deg_kernel()


NBUF = 2
HCPT = CPT // 2
HNITER = HCPT // NBUF


def _agg_body(y_hbm, src2d_hbm, dst2d_hbm, out_hbm,
              srcbuf, dstbuf, rows0, rows1,
              acc, gs0, gs1, ss0, ss1):
    rows = (rows0, rows1)
    gsem = (gs0, gs1)
    ssem = (ss0, ss1)
    c = lax.axis_index("c")
    s = lax.axis_index("s")
    wid = c * NS + s

    @pl.when(c == 0)
    def _():
        pltpu.sync_copy(y_hbm.at[pl.ds(s * SPAN, SPAN)],
                        acc.at[pl.ds(s * SPAN, SPAN)])

    @pl.when(c != 0)
    def _():
        @pl.loop(0, CH)
        def _(r):
            for k in range(D // 16):
                rows0[r, pl.ds(k * 16, 16)] = jnp.zeros((16,), jnp.float32)

        for q in range(SPAN // CH):
            pltpu.sync_copy(rows0, acc.at[pl.ds(s * SPAN + q * CH, CH)])

    plsc.subcore_barrier()

    ACT = 0
    for p in range(2 * NC):
      @pl.when(c == ACT)
      def _():
        base = s * (2 * NC * HCPT) + p * HCPT
        pltpu.sync_copy(src2d_hbm.at[pl.ds(base, HCPT)], srcbuf)
        pltpu.sync_copy(dst2d_hbm.at[pl.ds(base, HCPT)], dstbuf)
        for b in range(NBUF):
            pltpu.async_copy(y_hbm.at[srcbuf.at[b]], rows[b], gsem[b])

        @pl.loop(0, HNITER)
        def _(jj):
            for b in range(NBUF):
                j = jj * NBUF + b
                pltpu.make_async_copy(y_hbm.at[srcbuf.at[j]], rows[b],
                                      gsem[b]).wait()
                pltpu.async_copy(rows[b], acc.at[dstbuf.at[j]], ssem[b],
                                 add=True)

                @pl.when(jj < HNITER - 1)
                def _():
                    pltpu.make_async_copy(rows[b], acc.at[dstbuf.at[j]],
                                          ssem[b]).wait()
                    pltpu.async_copy(y_hbm.at[srcbuf.at[j + NBUF]], rows[b],
                                     gsem[b])

        for b in range(NBUF):
            pltpu.make_async_copy(rows[b], acc.at[dstbuf.at[HCPT - NBUF + b]],
                                  ssem[b]).wait()

    plsc.subcore_barrier()
    pltpu.sync_copy(
        acc.at[pl.ds(s * SPAN, SPAN)], out_hbm.at[c, pl.ds(s * SPAN, SPAN)]
    )


def _make_agg_kernel(interpret=False):
    return functools.partial(
        pl.kernel,
        out_type=jax.ShapeDtypeStruct((NC, N_ACC, D), jnp.float32),
        mesh=_MESH,
        scratch_types=[
            pltpu.VMEM((HCPT, CH), jnp.int32),
            pltpu.VMEM((HCPT, CH), jnp.int32),
            pltpu.VMEM((CH, D), jnp.float32),
            pltpu.VMEM((CH, D), jnp.float32),
            pltpu.VMEM_SHARED((N_ACC, D), jnp.float32),
            pltpu.SemaphoreType.DMA,
            pltpu.SemaphoreType.DMA,
            pltpu.SemaphoreType.DMA,
            pltpu.SemaphoreType.DMA,
        ],
        interpret=interpret,
    )(_agg_body)


_agg_kernel = _make_agg_kernel()


def _mm_body(x_ref, w_ref, d0_ref, d1_ref, y_ref):
    deg = d0_ref[...] + d1_ref[...] + 1.0
    dis = lax.rsqrt(deg)
    y_ref[...] = dis * jnp.dot(
        x_ref[...], w_ref[...], preferred_element_type=jnp.float32
    )


def _fin_body(p0_ref, p1_ref, d0_ref, d1_ref, b_ref, o_ref):
    dis = lax.rsqrt(d0_ref[...] + d1_ref[...] + 1.0)
    o_ref[...] = dis * (p0_ref[...] + p1_ref[...]) + b_ref[...]


def kernel(x, edge_index, W, b):
    ei = edge_index.astype(jnp.int32)
    pad = E_PAD - E
    src_p = jnp.concatenate([ei[0], jnp.zeros((pad,), jnp.int32)])
    dst_p = jnp.concatenate([ei[1], jnp.full((pad,), N, jnp.int32)])
    src2d = src_p.reshape(NW * CPT, CH)
    dst2d = dst_p.reshape(NW * CPT, CH)

    x_p = jnp.pad(x, ((0, N_ACC - N), (0, 0)))

    degout = _deg_kernel(dst2d)
    d0 = degout[0, :, None]
    d1 = degout[1, :, None]

    y = pl.pallas_call(
        _mm_body,
        out_shape=jax.ShapeDtypeStruct((N_ACC, D), jnp.float32),
    )(x_p, W, d0, d1)

    parts = _agg_kernel(y, src2d, dst2d)

    out = pl.pallas_call(
        _fin_body,
        out_shape=jax.ShapeDtypeStruct((N, D), jnp.float32),
    )(parts[0, :N], parts[1, :N], d0[:N], d1[:N], b.reshape(1, D))
    return out


# speedup vs baseline: 1.0611x; 1.0611x over previous
import functools

import jax
import jax.numpy as jnp
from jax import lax
from jax.experimental import pallas as pl
from jax.experimental.pallas import tpu as pltpu
from jax.experimental.pallas import tpu_sc as plsc

N = 10000
E = 320000
D = 128

NC = 2
NS = 16
NW = NC * NS

CH = 128
CPT = 80
EPT = CPT * CH
E_PAD = NW * EPT
N_ACC = 10240
SPAN = N_ACC // NS
DW = 16

_MESH = plsc.VectorSubcoreMesh(
    core_axis_name="c", subcore_axis_name="s", num_cores=NC, num_subcores=NS
)


def _deg_body(dst2d_hbm, degout_hbm, idxbuf, ones_v, zseg, acc):
    c = lax.axis_index("c")
    s = lax.axis_index("s")
    wid = c * NS + s

    def z(k, carry):
        zseg[pl.ds(k * 16, 16)] = jnp.zeros((16,), jnp.float32)
        return carry

    lax.fori_loop(0, SPAN // 16, z, 0)
    pltpu.sync_copy(zseg, acc.at[pl.ds(s * SPAN, SPAN)])

    def o(k, carry):
        ones_v[pl.ds(k * 16, 16)] = jnp.ones((16,), jnp.float32)
        return carry

    lax.fori_loop(0, CH // 16, o, 0)
    pltpu.sync_copy(dst2d_hbm.at[pl.ds(wid * CPT, CPT)], idxbuf)
    plsc.subcore_barrier()

    def body(j, carry):
        pltpu.sync_copy(ones_v, acc.at[idxbuf.at[j]], add=True)
        return carry

    lax.fori_loop(0, CPT, body, 0)
    plsc.subcore_barrier()
    pltpu.sync_copy(
        acc.at[pl.ds(s * SPAN, SPAN)], degout_hbm.at[c, pl.ds(s * SPAN, SPAN)]
    )


def _make_deg_kernel(interpret=False):
    return functools.partial(
        pl.kernel,
        out_type=jax.ShapeDtypeStruct((NC, N_ACC), jnp.float32),
        mesh=_MESH,
        scratch_types=[
            pltpu.VMEM((CPT, CH), jnp.int32),
            pltpu.VMEM((CH,), jnp.float32),
            pltpu.VMEM((SPAN,), jnp.float32),
            pltpu.VMEM_SHARED((N_ACC,), jnp.float32),
        ],
        interpret=interpret,
    )(_deg_body)


_deg_kernel = _make_deg_kernel()


NBUF = 2
HCPT = CPT // 2
HNITER = HCPT // NBUF


def _agg_body(y_hbm, src2d_hbm, dst2d_hbm, out_hbm,
              srcbuf, dstbuf, rows0, rows1,
              acc, gs0, gs1, ss0, ss1):
    rows = (rows0, rows1)
    gsem = (gs0, gs1)
    ssem = (ss0, ss1)
    c = lax.axis_index("c")
    s = lax.axis_index("s")
    wid = c * NS + s

    @pl.when(c == 0)
    def _():
        pltpu.sync_copy(y_hbm.at[pl.ds(s * SPAN, SPAN)],
                        acc.at[pl.ds(s * SPAN, SPAN)])

    @pl.when(c != 0)
    def _():
        @pl.loop(0, CH)
        def _(r):
            for k in range(D // 16):
                rows0[r, pl.ds(k * 16, 16)] = jnp.zeros((16,), jnp.float32)

        for q in range(SPAN // CH):
            pltpu.sync_copy(rows0, acc.at[pl.ds(s * SPAN + q * CH, CH)])

    plsc.subcore_barrier()

    ACT = 1
    for p in range(2 * NC):
      @pl.when(c == ACT)
      def _():
        base = s * (2 * NC * HCPT) + p * HCPT
        pltpu.sync_copy(src2d_hbm.at[pl.ds(base, HCPT)], srcbuf)
        pltpu.sync_copy(dst2d_hbm.at[pl.ds(base, HCPT)], dstbuf)
        for b in range(NBUF):
            pltpu.async_copy(y_hbm.at[srcbuf.at[b]], rows[b], gsem[b])

        @pl.loop(0, HNITER)
        def _(jj):
            for b in range(NBUF):
                j = jj * NBUF + b
                pltpu.make_async_copy(y_hbm.at[srcbuf.at[j]], rows[b],
                                      gsem[b]).wait()
                pltpu.async_copy(rows[b], acc.at[dstbuf.at[j]], ssem[b],
                                 add=True)

                @pl.when(jj < HNITER - 1)
                def _():
                    pltpu.make_async_copy(rows[b], acc.at[dstbuf.at[j]],
                                          ssem[b]).wait()
                    pltpu.async_copy(y_hbm.at[srcbuf.at[j + NBUF]], rows[b],
                                     gsem[b])

        for b in range(NBUF):
            pltpu.make_async_copy(rows[b], acc.at[dstbuf.at[HCPT - NBUF + b]],
                                  ssem[b]).wait()

    plsc.subcore_barrier()
    pltpu.sync_copy(
        acc.at[pl.ds(s * SPAN, SPAN)], out_hbm.at[c, pl.ds(s * SPAN, SPAN)]
    )


def _make_agg_kernel(interpret=False):
    return functools.partial(
        pl.kernel,
        out_type=jax.ShapeDtypeStruct((NC, N_ACC, D), jnp.float32),
        mesh=_MESH,
        scratch_types=[
            pltpu.VMEM((HCPT, CH), jnp.int32),
            pltpu.VMEM((HCPT, CH), jnp.int32),
            pltpu.VMEM((CH, D), jnp.float32),
            pltpu.VMEM((CH, D), jnp.float32),
            pltpu.VMEM_SHARED((N_ACC, D), jnp.float32),
            pltpu.SemaphoreType.DMA,
            pltpu.SemaphoreType.DMA,
            pltpu.SemaphoreType.DMA,
            pltpu.SemaphoreType.DMA,
        ],
        interpret=interpret,
    )(_agg_body)


_agg_kernel = _make_agg_kernel()


def _mm_body(x_ref, w_ref, d0_ref, d1_ref, y_ref):
    deg = d0_ref[...] + d1_ref[...] + 1.0
    dis = lax.rsqrt(deg)
    y_ref[...] = dis * jnp.dot(
        x_ref[...], w_ref[...], preferred_element_type=jnp.float32
    )


def _fin_body(p0_ref, p1_ref, d0_ref, d1_ref, b_ref, o_ref):
    dis = lax.rsqrt(d0_ref[...] + d1_ref[...] + 1.0)
    o_ref[...] = dis * (p0_ref[...] + p1_ref[...]) + b_ref[...]


def kernel(x, edge_index, W, b):
    ei = edge_index.astype(jnp.int32)
    pad = E_PAD - E
    src_p = jnp.concatenate([ei[0], jnp.zeros((pad,), jnp.int32)])
    dst_p = jnp.concatenate([ei[1], jnp.full((pad,), N, jnp.int32)])
    src2d = src_p.reshape(NW * CPT, CH)
    dst2d = dst_p.reshape(NW * CPT, CH)

    x_p = jnp.pad(x, ((0, N_ACC - N), (0, 0)))

    degout = _deg_kernel(dst2d)
    d0 = degout[0, :, None]
    d1 = degout[1, :, None]

    y = pl.pallas_call(
        _mm_body,
        out_shape=jax.ShapeDtypeStruct((N_ACC, D), jnp.float32),
    )(x_p, W, d0, d1)

    parts = _agg_kernel(y, src2d, dst2d)

    out = pl.pallas_call(
        _fin_body,
        out_shape=jax.ShapeDtypeStruct((N, D), jnp.float32),
    )(parts[0, :N], parts[1, :N], d0[:N], d1[:N], b.reshape(1, D))
    return out


# speedup vs baseline: 1.1877x; 1.1193x over previous
import functools

import jax
import jax.numpy as jnp
from jax import lax
from jax.experimental import pallas as pl
from jax.experimental.pallas import tpu as pltpu
from jax.experimental.pallas import tpu_sc as plsc

N = 10000
E = 320000
D = 128

NC = 2
NS = 16
NW = NC * NS

CH = 128
CPT = 80
EPT = CPT * CH
E_PAD = NW * EPT
N_ACC = 10240
SPAN = N_ACC // NS
DW = 16

_MESH = plsc.VectorSubcoreMesh(
    core_axis_name="c", subcore_axis_name="s", num_cores=NC, num_subcores=NS
)


def _deg_body(dst2d_hbm, degout_hbm, idxbuf, ones_v, zseg, acc):
    c = lax.axis_index("c")
    s = lax.axis_index("s")
    wid = c * NS + s

    def z(k, carry):
        zseg[pl.ds(k * 16, 16)] = jnp.zeros((16,), jnp.float32)
        return carry

    lax.fori_loop(0, SPAN // 16, z, 0)
    pltpu.sync_copy(zseg, acc.at[pl.ds(s * SPAN, SPAN)])

    def o(k, carry):
        ones_v[pl.ds(k * 16, 16)] = jnp.ones((16,), jnp.float32)
        return carry

    lax.fori_loop(0, CH // 16, o, 0)
    pltpu.sync_copy(dst2d_hbm.at[pl.ds(wid * CPT, CPT)], idxbuf)
    plsc.subcore_barrier()

    def body(j, carry):
        pltpu.sync_copy(ones_v, acc.at[idxbuf.at[j]], add=True)
        return carry

    lax.fori_loop(0, CPT, body, 0)
    plsc.subcore_barrier()
    pltpu.sync_copy(
        acc.at[pl.ds(s * SPAN, SPAN)], degout_hbm.at[c, pl.ds(s * SPAN, SPAN)]
    )


def _make_deg_kernel(interpret=False):
    return functools.partial(
        pl.kernel,
        out_type=jax.ShapeDtypeStruct((NC, N_ACC), jnp.float32),
        mesh=_MESH,
        scratch_types=[
            pltpu.VMEM((CPT, CH), jnp.int32),
            pltpu.VMEM((CH,), jnp.float32),
            pltpu.VMEM((SPAN,), jnp.float32),
            pltpu.VMEM_SHARED((N_ACC,), jnp.float32),
        ],
        interpret=interpret,
    )(_deg_body)


_deg_kernel = _make_deg_kernel()


NBUF = 2
HCPT = CPT // 2
HNITER = HCPT // NBUF


def _agg_body(y_hbm, src2d_hbm, dst2d_hbm, out_hbm,
              srcbuf, dstbuf, rows0, rows1,
              acc, gs0, gs1, ss0, ss1):
    rows = (rows0, rows1)
    gsem = (gs0, gs1)
    ssem = (ss0, ss1)
    c = lax.axis_index("c")
    s = lax.axis_index("s")
    wid = c * NS + s

    @pl.when(c == 0)
    def _():
        pltpu.sync_copy(y_hbm.at[pl.ds(s * SPAN, SPAN)],
                        acc.at[pl.ds(s * SPAN, SPAN)])

    @pl.when(c != 0)
    def _():
        @pl.loop(0, CH)
        def _(r):
            for k in range(D // 16):
                rows0[r, pl.ds(k * 16, 16)] = jnp.zeros((16,), jnp.float32)

        for q in range(SPAN // CH):
            pltpu.sync_copy(rows0, acc.at[pl.ds(s * SPAN + q * CH, CH)])

    plsc.subcore_barrier()

    for p in range(2):
        base = wid * CPT + p * HCPT
        pltpu.sync_copy(src2d_hbm.at[pl.ds(base, HCPT)], srcbuf)
        pltpu.sync_copy(dst2d_hbm.at[pl.ds(base, HCPT)], dstbuf)
        for b in range(NBUF):
            pltpu.async_copy(y_hbm.at[srcbuf.at[b]], rows[b], gsem[b])

        @pl.loop(0, HNITER)
        def _(jj):
            for b in range(NBUF):
                j = jj * NBUF + b
                pltpu.make_async_copy(y_hbm.at[srcbuf.at[j]], rows[b],
                                      gsem[b]).wait()
                pltpu.async_copy(rows[b], acc.at[dstbuf.at[j]], ssem[b],
                                 add=True)

                @pl.when(jj < HNITER - 1)
                def _():
                    pltpu.make_async_copy(rows[b], acc.at[dstbuf.at[j]],
                                          ssem[b]).wait()
                    pltpu.async_copy(y_hbm.at[srcbuf.at[j + NBUF]], rows[b],
                                     gsem[b])

        for b in range(NBUF):
            pltpu.make_async_copy(rows[b], acc.at[dstbuf.at[HCPT - NBUF + b]],
                                  ssem[b]).wait()

    plsc.subcore_barrier()
    pltpu.sync_copy(
        acc.at[pl.ds(s * SPAN, SPAN)], out_hbm.at[c, pl.ds(s * SPAN, SPAN)]
    )


def _make_agg_kernel(interpret=False):
    return functools.partial(
        pl.kernel,
        out_type=jax.ShapeDtypeStruct((NC, N_ACC, D), jnp.float32),
        mesh=_MESH,
        scratch_types=[
            pltpu.VMEM((HCPT, CH), jnp.int32),
            pltpu.VMEM((HCPT, CH), jnp.int32),
            pltpu.VMEM((CH, D), jnp.float32),
            pltpu.VMEM((CH, D), jnp.float32),
            pltpu.VMEM_SHARED((N_ACC, D), jnp.float32),
            pltpu.SemaphoreType.DMA,
            pltpu.SemaphoreType.DMA,
            pltpu.SemaphoreType.DMA,
            pltpu.SemaphoreType.DMA,
        ],
        interpret=interpret,
    )(_agg_body)


_agg_kernel = _make_agg_kernel()


def _mm_body(x_ref, w_ref, d0_ref, d1_ref, y_ref):
    deg = d0_ref[...] + d1_ref[...] + 1.0
    dis = lax.rsqrt(deg)
    y_ref[...] = dis * jnp.dot(
        x_ref[...], w_ref[...], preferred_element_type=jnp.float32
    )


def _fin_body(p0_ref, p1_ref, d0_ref, d1_ref, b_ref, o_ref):
    dis = lax.rsqrt(d0_ref[...] + d1_ref[...] + 1.0)
    o_ref[...] = dis * (p0_ref[...] + p1_ref[...]) + b_ref[...]


def kernel(x, edge_index, W, b):
    ei = edge_index.astype(jnp.int32)
    pad = E_PAD - E
    src_p = jnp.concatenate([ei[0], jnp.zeros((pad,), jnp.int32)])
    dst_p = jnp.concatenate([ei[1], jnp.full((pad,), N, jnp.int32)])
    src2d = src_p.reshape(NW * CPT, CH)
    dst2d = dst_p.reshape(NW * CPT, CH)

    x_p = jnp.pad(x, ((0, N_ACC - N), (0, 0)))

    degout = _deg_kernel(dst2d)
    d0 = degout[0, :, None]
    d1 = degout[1, :, None]

    y = pl.pallas_call(
        _mm_body,
        out_shape=jax.ShapeDtypeStruct((N_ACC, D), jnp.float32),
    )(x_p, W, d0, d1)

    parts = _agg_kernel(y, src2d, dst2d)

    out = pl.pallas_call(
        _fin_body,
        out_shape=jax.ShapeDtypeStruct((N, D), jnp.float32),
    )(parts[0, :N], parts[1, :N], d0[:N], d1[:N], b.reshape(1, D))
    return out


# speedup vs baseline: 1.2160x; 1.0238x over previous
import functools

import jax
import jax.numpy as jnp
from jax import lax
from jax.experimental import pallas as pl
from jax.experimental.pallas import tpu as pltpu
from jax.experimental.pallas import tpu_sc as plsc

N = 10000
E = 320000
D = 128

NC = 2
NS = 16
NW = NC * NS

CH = 128
CPT = 80
EPT = CPT * CH
E_PAD = NW * EPT
N_ACC = 10240
SPAN = N_ACC // NS
DW = 16

_MESH = plsc.VectorSubcoreMesh(
    core_axis_name="c", subcore_axis_name="s", num_cores=NC, num_subcores=NS
)


def _deg_body(dst2d_hbm, degout_hbm, idxbuf, ones_v, zseg, acc):
    c = lax.axis_index("c")
    s = lax.axis_index("s")
    wid = c * NS + s

    def z(k, carry):
        zseg[pl.ds(k * 16, 16)] = jnp.zeros((16,), jnp.float32)
        return carry

    lax.fori_loop(0, SPAN // 16, z, 0)
    pltpu.sync_copy(zseg, acc.at[pl.ds(s * SPAN, SPAN)])

    def o(k, carry):
        ones_v[pl.ds(k * 16, 16)] = jnp.ones((16,), jnp.float32)
        return carry

    lax.fori_loop(0, CH // 16, o, 0)
    pltpu.sync_copy(dst2d_hbm.at[pl.ds(wid * CPT, CPT)], idxbuf)
    plsc.subcore_barrier()

    def body(j, carry):
        pltpu.sync_copy(ones_v, acc.at[idxbuf.at[j]], add=True)
        return carry

    lax.fori_loop(0, CPT, body, 0)
    plsc.subcore_barrier()
    pltpu.sync_copy(
        acc.at[pl.ds(s * SPAN, SPAN)], degout_hbm.at[c, pl.ds(s * SPAN, SPAN)]
    )


def _make_deg_kernel(interpret=False):
    return functools.partial(
        pl.kernel,
        out_type=jax.ShapeDtypeStruct((NC, N_ACC), jnp.float32),
        mesh=_MESH,
        scratch_types=[
            pltpu.VMEM((CPT, CH), jnp.int32),
            pltpu.VMEM((CH,), jnp.float32),
            pltpu.VMEM((SPAN,), jnp.float32),
            pltpu.VMEM_SHARED((N_ACC,), jnp.float32),
        ],
        interpret=interpret,
    )(_deg_body)


_deg_kernel = _make_deg_kernel()


NBUF = 2
HCPT = CPT // 2
HNITER = HCPT // NBUF


def _agg_body(y_hbm, src2d_hbm, dst2d_hbm, out_hbm,
              srcbuf, dstbuf, rows0, rows1,
              acc, gs0, gs1, ss0, ss1):
    rows = (rows0, rows1)
    gsem = (gs0, gs1)
    ssem = (ss0, ss1)
    c = lax.axis_index("c")
    s = lax.axis_index("s")
    wid = c * NS + s

    @pl.when(c == 0)
    def _():
        pltpu.sync_copy(y_hbm.at[pl.ds(s * SPAN, SPAN)],
                        acc.at[pl.ds(s * SPAN, SPAN)])

    @pl.when(c != 0)
    def _():
        @pl.loop(0, CH)
        def _(r):
            for k in range(D // 16):
                rows0[r, pl.ds(k * 16, 16)] = jnp.zeros((16,), jnp.float32)

        for q in range(SPAN // CH):
            pltpu.sync_copy(rows0, acc.at[pl.ds(s * SPAN + q * CH, CH)])

    plsc.subcore_barrier()

    for p in range(2):
        base = wid * CPT + p * HCPT
        pltpu.sync_copy(src2d_hbm.at[pl.ds(base, HCPT)], srcbuf)
        pltpu.sync_copy(dst2d_hbm.at[pl.ds(base, HCPT)], dstbuf)
        for b in range(NBUF):
            pltpu.async_copy(y_hbm.at[srcbuf.at[b]], rows[b], gsem[b])

        @pl.loop(0, HNITER)
        def _(jj):
            for b in range(NBUF):
                j = jj * NBUF + b
                pltpu.make_async_copy(y_hbm.at[srcbuf.at[j]], rows[b],
                                      gsem[b]).wait()
                pltpu.async_copy(rows[b], acc.at[dstbuf.at[j]], ssem[b],
                                 add=True)

                @pl.when(jj < HNITER - 1)
                def _():
                    pltpu.make_async_copy(rows[b], acc.at[dstbuf.at[j]],
                                          ssem[b]).wait()
                    pltpu.async_copy(y_hbm.at[srcbuf.at[j + NBUF]], rows[b],
                                     gsem[b])

        for b in range(NBUF):
            pltpu.make_async_copy(rows[b], acc.at[dstbuf.at[HCPT - NBUF + b]],
                                  ssem[b]).wait()

    plsc.subcore_barrier()
    pltpu.sync_copy(
        acc.at[pl.ds(s * SPAN, SPAN)], out_hbm.at[c, pl.ds(s * SPAN, SPAN)]
    )


def _make_agg_kernel(interpret=False):
    return functools.partial(
        pl.kernel,
        out_type=jax.ShapeDtypeStruct((NC, N_ACC, D), jnp.float32),
        mesh=_MESH,
        scratch_types=[
            pltpu.VMEM((HCPT, CH), jnp.int32),
            pltpu.VMEM((HCPT, CH), jnp.int32),
            pltpu.VMEM((CH, D), jnp.float32),
            pltpu.VMEM((CH, D), jnp.float32),
            pltpu.VMEM_SHARED((N_ACC, D), jnp.float32),
            pltpu.SemaphoreType.DMA,
            pltpu.SemaphoreType.DMA,
            pltpu.SemaphoreType.DMA,
            pltpu.SemaphoreType.DMA,
        ],
        interpret=interpret,
    )(_agg_body)


_agg_kernel = _make_agg_kernel()


def _mm_body(x_ref, w_ref, d0_ref, d1_ref, y_ref):
    deg = d0_ref[...] + d1_ref[...] + 1.0
    dis = lax.rsqrt(deg)
    y = dis * jnp.dot(
        x_ref[...], w_ref[...], preferred_element_type=jnp.float32
    )
    y_ref[pl.ds(0, N_ACC), :] = y
    y_ref[pl.ds(N_ACC, N_ACC), :] = y


def _fin_body(p0_ref, p1_ref, d0_ref, d1_ref, b_ref, o_ref):
    dis = lax.rsqrt(d0_ref[...] + d1_ref[...] + 1.0)
    o_ref[...] = dis * (p0_ref[...] + p1_ref[...]) + b_ref[...]


def kernel(x, edge_index, W, b):
    ei = edge_index.astype(jnp.int32)
    pad = E_PAD - E
    src_p = jnp.concatenate([ei[0], jnp.zeros((pad,), jnp.int32)])
    dst_p = jnp.concatenate([ei[1], jnp.full((pad,), N, jnp.int32)])
    src2d = src_p.reshape(NW * CPT, CH)
    dst2d = dst_p.reshape(NW * CPT, CH)
    row_core = (jnp.arange(NW * CPT, dtype=jnp.int32) >= NS * CPT)
    src2d = src2d + (row_core[:, None].astype(jnp.int32) * N_ACC)

    x_p = jnp.pad(x, ((0, N_ACC - N), (0, 0)))

    degout = _deg_kernel(dst2d)
    d0 = degout[0, :, None]
    d1 = degout[1, :, None]

    y = pl.pallas_call(
        _mm_body,
        out_shape=jax.ShapeDtypeStruct((2 * N_ACC, D), jnp.float32),
    )(x_p, W, d0, d1)

    parts = _agg_kernel(y, src2d, dst2d)

    out = pl.pallas_call(
        _fin_body,
        out_shape=jax.ShapeDtypeStruct((N, D), jnp.float32),
    )(parts[0, :N], parts[1, :N], d0[:N], d1[:N], b.reshape(1, D))
    return out
